# baseline (speedup 1.0000x reference)
import functools

import jax
import jax.numpy as jnp
from jax.experimental import pallas as pl
from jax.experimental.pallas import tpu as pltpu

_T = 128


def _fused_loss_kernel(ti_ref, ei_ref, sr_ref, er_ref, fi_ref,
                       hs_ref, cw_ref, cb_ref, wr_ref, br_ref,
                       cols_ref, rows_ref, out_ref):
    T = hs_ref.shape[0]
    E = cb_ref.shape[2]
    g = pl.program_id(0)

    @pl.when(g == 0)
    def _init():
        out_ref[0] = jnp.float32(0.0)

    x = hs_ref[...]
    c_iota = jax.lax.broadcasted_iota(jnp.int32, (T, E), 1)
    r_iota = jax.lax.broadcasted_iota(jnp.int32, (T, 1), 0)

    start = sr_ref[g]
    end = er_ref[g]

    @pl.when(end > start)
    def _col_part():
        w = cw_ref[0]
        logits = jnp.dot(x, w, preferred_element_type=jnp.float32) + cb_ref[0]
        m = jnp.max(logits, axis=1, keepdims=True)
        lse = jnp.log(jnp.sum(jnp.exp(logits - m), axis=1, keepdims=True)) + m
        tgt = jnp.sum(jnp.where(c_iota == cols_ref[:, 0:1], logits, 0.0),
                      axis=1, keepdims=True)
        active = (r_iota >= start) & (r_iota < end)
        out_ref[0] += jnp.sum(jnp.where(active, lse - tgt, 0.0))

    @pl.when(fi_ref[g] == 1)
    def _row_part():
        logits = jnp.dot(x, wr_ref[...], preferred_element_type=jnp.float32) + br_ref[...]
        m = jnp.max(logits, axis=1, keepdims=True)
        lse = jnp.log(jnp.sum(jnp.exp(logits - m), axis=1, keepdims=True)) + m
        tgt = jnp.sum(jnp.where(c_iota == rows_ref[:, 0:1], logits, 0.0),
                      axis=1, keepdims=True)
        out_ref[0] += jnp.sum(lse - tgt)


def kernel(hidden_states, target_ids, W_row, b_row, col_weight, col_bias):
    E, D = W_row.shape
    Bb, S, _ = hidden_states.shape
    N = Bb * S
    T = _T
    num_tiles = N // T
    G = num_tiles + E

    ids = target_ids.reshape(-1).astype(jnp.int32)
    row_ids = ids // E
    col_ids = ids % E

    sort_idx = jnp.argsort(row_ids).astype(jnp.int32)
    row_sorted = jnp.take(row_ids, sort_idx)
    col_sorted = jnp.take(col_ids, sort_idx)
    hs_flat = hidden_states.reshape(N, D)
    hs_sorted = jnp.take(hs_flat, sort_idx, axis=0).astype(jnp.bfloat16)

    counts = jnp.zeros((E,), jnp.int32).at[row_ids].add(1)
    offsets = (jnp.cumsum(counts) - counts).astype(jnp.int32)
    tile_starts = jnp.arange(num_tiles, dtype=jnp.int32) * T
    seg_starts = jnp.sort(jnp.concatenate([tile_starts, offsets]))
    seg_ends = jnp.concatenate([seg_starts[1:], jnp.array([N], jnp.int32)])
    tile_of = jnp.minimum(seg_starts // T, num_tiles - 1)
    expert_of = jnp.take(row_sorted, jnp.minimum(seg_starts, N - 1))
    start_rel = seg_starts - tile_of * T
    end_rel = seg_ends - tile_of * T
    first = ((start_rel == 0) & (end_rel > 0)).astype(jnp.int32)

    cols_bcast = jnp.broadcast_to(col_sorted[:, None], (N, 128))
    rows_bcast = jnp.broadcast_to(row_sorted[:, None], (N, 128))

    w_row_t = W_row.T.astype(jnp.bfloat16)
    b_row2 = b_row.reshape(1, E)
    cb3 = col_bias.reshape(E, 1, E)
    cw_bf = col_weight.astype(jnp.bfloat16)

    grid_spec = pltpu.PrefetchScalarGridSpec(
        num_scalar_prefetch=5,
        grid=(G,),
        in_specs=[
            pl.BlockSpec((T, D), lambda g, ti, ei, sr, er, fi: (ti[g], 0)),
            pl.BlockSpec((1, D, E), lambda g, ti, ei, sr, er, fi: (ei[g], 0, 0)),
            pl.BlockSpec((1, 1, E), lambda g, ti, ei, sr, er, fi: (ei[g], 0, 0)),
            pl.BlockSpec((D, E), lambda g, ti, ei, sr, er, fi: (0, 0)),
            pl.BlockSpec((1, E), lambda g, ti, ei, sr, er, fi: (0, 0)),
            pl.BlockSpec((T, 128), lambda g, ti, ei, sr, er, fi: (ti[g], 0)),
            pl.BlockSpec((T, 128), lambda g, ti, ei, sr, er, fi: (ti[g], 0)),
        ],
        out_specs=pl.BlockSpec((1,), lambda g, ti, ei, sr, er, fi: (0,),
                               memory_space=pltpu.SMEM),
    )

    total = pl.pallas_call(
        _fused_loss_kernel,
        grid_spec=grid_spec,
        out_shape=jax.ShapeDtypeStruct((1,), jnp.float32),
    )(tile_of, expert_of, start_rel, end_rel, first,
      hs_sorted, cw_bf, cb3, w_row_t, b_row2,
      cols_bcast, rows_bcast)

    return total[0] / jnp.float32(N)


# speedup vs baseline: 1.6757x; 1.6757x over previous
import functools

import jax
import jax.numpy as jnp
from jax.experimental import pallas as pl
from jax.experimental.pallas import tpu as pltpu

_T = 128


def _fused_loss_kernel(ti_ref, ei_ref, sr_ref, er_ref, fi_ref,
                       hs_ref, cw_ref, cb_ref, wr_ref, br_ref,
                       cols_ref, rows_ref, out_ref):
    T = hs_ref.shape[0]
    E = cb_ref.shape[2]
    g = pl.program_id(0)

    @pl.when(g == 0)
    def _init():
        out_ref[0] = jnp.float32(0.0)

    x = hs_ref[...].astype(jnp.bfloat16)
    c_iota = jax.lax.broadcasted_iota(jnp.int32, (T, E), 1)
    r_iota = jax.lax.broadcasted_iota(jnp.int32, (T, 1), 0)

    start = sr_ref[g]
    end = er_ref[g]

    @pl.when(end > start)
    def _col_part():
        w = cw_ref[0].astype(jnp.bfloat16)
        logits = jnp.dot(x, w, preferred_element_type=jnp.float32) + cb_ref[0]
        m = jnp.max(logits, axis=1, keepdims=True)
        lse = jnp.log(jnp.sum(jnp.exp(logits - m), axis=1, keepdims=True)) + m
        tgt = jnp.sum(jnp.where(c_iota == cols_ref[:, 0:1], logits, 0.0),
                      axis=1, keepdims=True)
        active = (r_iota >= start) & (r_iota < end)
        out_ref[0] += jnp.sum(jnp.where(active, lse - tgt, 0.0))

    @pl.when(fi_ref[g] == 1)
    def _row_part():
        logits = jnp.dot(x, wr_ref[...].astype(jnp.bfloat16),
                         preferred_element_type=jnp.float32) + br_ref[...]
        m = jnp.max(logits, axis=1, keepdims=True)
        lse = jnp.log(jnp.sum(jnp.exp(logits - m), axis=1, keepdims=True)) + m
        tgt = jnp.sum(jnp.where(c_iota == rows_ref[:, 0:1], logits, 0.0),
                      axis=1, keepdims=True)
        out_ref[0] += jnp.sum(lse - tgt)


def kernel(hidden_states, target_ids, W_row, b_row, col_weight, col_bias):
    E, D = W_row.shape
    Bb, S, _ = hidden_states.shape
    N = Bb * S
    T = _T
    num_tiles = N // T
    G = num_tiles + E

    ids = target_ids.reshape(-1).astype(jnp.int32)
    row_ids = ids // E
    col_ids = ids % E

    sort_idx = jnp.argsort(row_ids).astype(jnp.int32)
    row_sorted = jnp.take(row_ids, sort_idx)
    col_sorted = jnp.take(col_ids, sort_idx)
    hs_flat = hidden_states.reshape(N, D)
    hs_sorted = jnp.take(hs_flat, sort_idx, axis=0)

    counts = jnp.zeros((E,), jnp.int32).at[row_ids].add(1)
    offsets = (jnp.cumsum(counts) - counts).astype(jnp.int32)
    tile_starts = jnp.arange(num_tiles, dtype=jnp.int32) * T
    seg_starts = jnp.sort(jnp.concatenate([tile_starts, offsets]))
    seg_ends = jnp.concatenate([seg_starts[1:], jnp.array([N], jnp.int32)])
    tile_of = jnp.minimum(seg_starts // T, num_tiles - 1)
    expert_of = jnp.take(row_sorted, jnp.minimum(seg_starts, N - 1))
    start_rel = seg_starts - tile_of * T
    end_rel = seg_ends - tile_of * T
    first = ((start_rel == 0) & (end_rel > 0)).astype(jnp.int32)

    cols_bcast = jnp.broadcast_to(col_sorted[:, None], (N, 128))
    rows_bcast = jnp.broadcast_to(row_sorted[:, None], (N, 128))

    w_row_t = W_row.T
    b_row2 = b_row.reshape(1, E)
    cb3 = col_bias.reshape(E, 1, E)

    grid_spec = pltpu.PrefetchScalarGridSpec(
        num_scalar_prefetch=5,
        grid=(G,),
        in_specs=[
            pl.BlockSpec((T, D), lambda g, ti, ei, sr, er, fi: (ti[g], 0)),
            pl.BlockSpec((1, D, E), lambda g, ti, ei, sr, er, fi: (ei[g], 0, 0)),
            pl.BlockSpec((1, 1, E), lambda g, ti, ei, sr, er, fi: (ei[g], 0, 0)),
            pl.BlockSpec((D, E), lambda g, ti, ei, sr, er, fi: (0, 0)),
            pl.BlockSpec((1, E), lambda g, ti, ei, sr, er, fi: (0, 0)),
            pl.BlockSpec((T, 128), lambda g, ti, ei, sr, er, fi: (ti[g], 0)),
            pl.BlockSpec((T, 128), lambda g, ti, ei, sr, er, fi: (ti[g], 0)),
        ],
        out_specs=pl.BlockSpec((1,), lambda g, ti, ei, sr, er, fi: (0,),
                               memory_space=pltpu.SMEM),
    )

    total = pl.pallas_call(
        _fused_loss_kernel,
        grid_spec=grid_spec,
        out_shape=jax.ShapeDtypeStruct((1,), jnp.float32),
    )(tile_of, expert_of, start_rel, end_rel, first,
      hs_sorted, col_weight, cb3, w_row_t, b_row2,
      cols_bcast, rows_bcast)

    return total[0] / jnp.float32(N)


# speedup vs baseline: 1.8479x; 1.1028x over previous
import functools

import jax
import jax.numpy as jnp
from jax.experimental import pallas as pl
from jax.experimental.pallas import tpu as pltpu

_T = 128


def _lse_rows(logits):
    m = jnp.max(logits, axis=1, keepdims=True)
    return jnp.log(jnp.sum(jnp.exp(logits - m), axis=1, keepdims=True)) + m


def _fused_loss_kernel(ti_ref, ei_ref, sr_ref, er_ref, la_ref,
                       hs_ref, cw_ref, cb_ref, wr_ref, br_ref,
                       cols_ref, rows_ref, out_ref, scratch_ref):
    T = hs_ref.shape[0]
    E = cb_ref.shape[2]
    g = pl.program_id(0)

    @pl.when(g == 0)
    def _init():
        out_ref[0] = jnp.float32(0.0)

    x = hs_ref[...].astype(jnp.bfloat16)
    r_iota = jax.lax.broadcasted_iota(jnp.int32, (T, 1), 0)

    start = sr_ref[g]
    end = er_ref[g]

    @pl.when(end > start)
    def _col_matmul():
        w = cw_ref[0].astype(jnp.bfloat16)
        logits = jnp.dot(x, w, preferred_element_type=jnp.float32) + cb_ref[0]
        active = (r_iota >= start) & (r_iota < end)
        scratch_ref[...] = jnp.where(active, logits, scratch_ref[...])

    @pl.when(la_ref[g] == 1)
    def _tile_epilogue():
        c_iota = jax.lax.broadcasted_iota(jnp.int32, (T, E), 1)
        cl = scratch_ref[...]
        c_lse = _lse_rows(cl)
        c_tgt = jnp.sum(jnp.where(c_iota == cols_ref[:, 0:1], cl, 0.0),
                        axis=1, keepdims=True)
        rl = jnp.dot(x, wr_ref[...].astype(jnp.bfloat16),
                     preferred_element_type=jnp.float32) + br_ref[...]
        r_lse = _lse_rows(rl)
        r_tgt = jnp.sum(jnp.where(c_iota == rows_ref[:, 0:1], rl, 0.0),
                        axis=1, keepdims=True)
        out_ref[0] += jnp.sum((c_lse - c_tgt) + (r_lse - r_tgt))


def kernel(hidden_states, target_ids, W_row, b_row, col_weight, col_bias):
    E, D = W_row.shape
    Bb, S, _ = hidden_states.shape
    N = Bb * S
    T = _T
    num_tiles = N // T
    G = num_tiles + E

    ids = target_ids.reshape(-1).astype(jnp.int32)
    row_ids = ids // E
    col_ids = ids % E

    sort_idx = jnp.argsort(row_ids).astype(jnp.int32)
    row_sorted = jnp.take(row_ids, sort_idx)
    col_sorted = jnp.take(col_ids, sort_idx)
    hs_flat = hidden_states.reshape(N, D)
    hs_sorted = jnp.take(hs_flat, sort_idx, axis=0)

    counts = jnp.zeros((E,), jnp.int32).at[row_ids].add(1)
    offsets = (jnp.cumsum(counts) - counts).astype(jnp.int32)
    tile_starts = jnp.arange(num_tiles, dtype=jnp.int32) * T
    seg_starts = jnp.sort(jnp.concatenate([tile_starts, offsets]))
    seg_ends = jnp.concatenate([seg_starts[1:], jnp.array([N], jnp.int32)])
    tile_of = jnp.minimum(seg_starts // T, num_tiles - 1)
    expert_of = jnp.take(row_sorted, jnp.minimum(seg_starts, N - 1))
    start_rel = seg_starts - tile_of * T
    end_rel = seg_ends - tile_of * T
    last = ((end_rel == T) & (end_rel > start_rel)).astype(jnp.int32)

    cols_bcast = jnp.broadcast_to(col_sorted[:, None], (N, 128))
    rows_bcast = jnp.broadcast_to(row_sorted[:, None], (N, 128))

    w_row_t = W_row.T
    b_row2 = b_row.reshape(1, E)
    cb3 = col_bias.reshape(E, 1, E)

    grid_spec = pltpu.PrefetchScalarGridSpec(
        num_scalar_prefetch=5,
        grid=(G,),
        in_specs=[
            pl.BlockSpec((T, D), lambda g, ti, ei, sr, er, la: (ti[g], 0)),
            pl.BlockSpec((1, D, E), lambda g, ti, ei, sr, er, la: (ei[g], 0, 0)),
            pl.BlockSpec((1, 1, E), lambda g, ti, ei, sr, er, la: (ei[g], 0, 0)),
            pl.BlockSpec((D, E), lambda g, ti, ei, sr, er, la: (0, 0)),
            pl.BlockSpec((1, E), lambda g, ti, ei, sr, er, la: (0, 0)),
            pl.BlockSpec((T, 128), lambda g, ti, ei, sr, er, la: (ti[g], 0)),
            pl.BlockSpec((T, 128), lambda g, ti, ei, sr, er, la: (ti[g], 0)),
        ],
        out_specs=pl.BlockSpec((1,), lambda g, ti, ei, sr, er, la: (0,),
                               memory_space=pltpu.SMEM),
        scratch_shapes=[pltpu.VMEM((T, E), jnp.float32)],
    )

    total = pl.pallas_call(
        _fused_loss_kernel,
        grid_spec=grid_spec,
        out_shape=jax.ShapeDtypeStruct((1,), jnp.float32),
    )(tile_of, expert_of, start_rel, end_rel, last,
      hs_sorted, col_weight, cb3, w_row_t, b_row2,
      cols_bcast, rows_bcast)

    return total[0] / jnp.float32(N)
